# baseline (speedup 1.0000x reference)
import functools

import jax
import jax.numpy as jnp
from jax import lax
from jax.experimental import pallas as pl
from jax.experimental.pallas import tpu as pltpu
from jax.experimental.pallas import tpu_sc as plsc

D_MODEL = 64
SCALE = 8.0
IDX_ROW = 128
CHUNK = 256
ROWS_PER_CHUNK = CHUNK // IDX_ROW


def _gather_scale(table, idx2d, B):
    info = plsc.get_sparse_core_info()
    n_workers = info.num_cores * info.num_subcores
    b_per_w = B // n_workers
    n_chunks = b_per_w // CHUNK
    idx_rows_per_w = b_per_w // IDX_ROW

    mesh = plsc.VectorSubcoreMesh(core_axis_name="c", subcore_axis_name="s")

    @functools.partial(
        pl.kernel,
        mesh=mesh,
        compiler_params=pltpu.CompilerParams(use_tc_tiling_on_sc=False),
        out_type=jax.ShapeDtypeStruct((B, 2 * D_MODEL), jnp.float32),
        scratch_types=[
            pltpu.VMEM((idx_rows_per_w, IDX_ROW), jnp.int32),
            pltpu.VMEM((2, CHUNK, D_MODEL), jnp.float32),
            pltpu.VMEM((2, CHUNK, 2 * D_MODEL), jnp.float32),
            pltpu.SemaphoreType.DMA,
            pltpu.SemaphoreType.DMA,
            pltpu.SemaphoreType.DMA,
            pltpu.SemaphoreType.DMA,
        ],
    )
    def k(table_hbm, idx_hbm, out_hbm, idx_v, rows_g, rows_o, sg0, sg1, so0, so1):
        wid = lax.axis_index("s") * info.num_cores + lax.axis_index("c")
        base = wid * b_per_w
        sems_g = (sg0, sg1)
        sems_o = (so0, so1)

        pltpu.sync_copy(idx_hbm.at[pl.ds(wid * idx_rows_per_w, idx_rows_per_w)],
                        idx_v)

        def start_gather(i, b):
            for j in range(ROWS_PER_CHUNK):
                pltpu.async_copy(
                    table_hbm.at[idx_v.at[i * ROWS_PER_CHUNK + j]],
                    rows_g.at[b, pl.ds(j * IDX_ROW, IDX_ROW)],
                    sems_g[b],
                )

        def wait_gather(b):
            for j in range(ROWS_PER_CHUNK):
                pltpu.make_async_copy(
                    table_hbm.at[idx_v.at[j]],
                    rows_g.at[b, pl.ds(j * IDX_ROW, IDX_ROW)],
                    sems_g[b],
                ).wait()

        def start_scatter(i, b):
            pltpu.async_copy(rows_o.at[b], out_hbm.at[pl.ds(base + i * CHUNK, CHUNK)],
                             sems_o[b])

        def wait_scatter(b):
            pltpu.make_async_copy(rows_o.at[b],
                                  out_hbm.at[pl.ds(base, CHUNK)],
                                  sems_o[b]).wait()

        def scale(b):
            def body8(t, c):
                r0 = t * 8
                for u in range(8):
                    for j in range(D_MODEL // 16):
                        sl = pl.ds(j * 16, 16)
                        rows_o[b, r0 + u, sl] = rows_g[b, r0 + u, sl] * SCALE
                return c

            lax.fori_loop(0, CHUNK // 8, body8, 0)

        start_gather(0, 0)
        start_gather(1, 1)

        for i in (0, 1):
            b = i % 2
            wait_gather(b)
            scale(b)
            start_scatter(i, b)
            start_gather(i + 2, b)

        def steady(t, c):
            i0 = 2 + t * 2
            for b in (0, 1):
                i = i0 + b
                wait_gather(b)
                wait_scatter(b)
                scale(b)
                start_scatter(i, b)
                start_gather(i + 2, b)
            return c

        lax.fori_loop(0, (n_chunks - 4) // 2, steady, 0)

        for i in (n_chunks - 2, n_chunks - 1):
            b = i % 2
            wait_gather(b)
            wait_scatter(b)
            scale(b)
            start_scatter(i, b)

        wait_scatter(0)
        wait_scatter(1)

    return k(table, idx2d)


def kernel(x, table):
    b, t = x.shape
    B = b * t
    idx2d = x.reshape(B // IDX_ROW, IDX_ROW).astype(jnp.int32)
    out = _gather_scale(table, idx2d, B)
    return out.reshape(b, t, 2 * D_MODEL)[:, :, :D_MODEL]


# speedup vs baseline: 1.4501x; 1.4501x over previous
import functools

import jax
import jax.numpy as jnp
from jax import lax
from jax.experimental import pallas as pl
from jax.experimental.pallas import tpu as pltpu
from jax.experimental.pallas import tpu_sc as plsc

D_MODEL = 64
SCALE = 8.0
IDX_ROW = 128
CHUNK = 256
ROWS_PER_CHUNK = CHUNK // IDX_ROW
OUT_W = 2 * D_MODEL


def _gather_scale(table2, idx2d, B):
    info = plsc.get_sparse_core_info()
    n_workers = info.num_cores * info.num_subcores
    b_per_w = B // n_workers
    n_chunks = b_per_w // CHUNK
    idx_rows_per_w = b_per_w // IDX_ROW

    mesh = plsc.VectorSubcoreMesh(core_axis_name="c", subcore_axis_name="s")

    @functools.partial(
        pl.kernel,
        mesh=mesh,
        compiler_params=pltpu.CompilerParams(use_tc_tiling_on_sc=False),
        out_type=jax.ShapeDtypeStruct((B, OUT_W), jnp.float32),
        scratch_types=[
            pltpu.VMEM((idx_rows_per_w, IDX_ROW), jnp.int32),
            pltpu.VMEM((2, CHUNK, D_MODEL), jnp.float32),
            pltpu.VMEM((2, CHUNK, D_MODEL), jnp.float32),
            pltpu.SemaphoreType.DMA,
            pltpu.SemaphoreType.DMA,
            pltpu.SemaphoreType.DMA,
            pltpu.SemaphoreType.DMA,
        ],
    )
    def k(table_hbm, idx_hbm, out_hbm, idx_v, rows_g, rows_o, sg0, sg1, so0, so1):
        wid = lax.axis_index("s") * info.num_cores + lax.axis_index("c")
        base = wid * b_per_w
        sems_g = (sg0, sg1)
        sems_o = (so0, so1)

        pltpu.sync_copy(idx_hbm.at[pl.ds(wid * idx_rows_per_w, idx_rows_per_w)],
                        idx_v)

        def start_gather(i, b):
            for j in range(ROWS_PER_CHUNK):
                pltpu.async_copy(
                    table_hbm.at[idx_v.at[i * ROWS_PER_CHUNK + j]],
                    rows_g.at[b, pl.ds(j * IDX_ROW, IDX_ROW)],
                    sems_g[b],
                )

        def wait_gather(b):
            for j in range(ROWS_PER_CHUNK):
                pltpu.make_async_copy(
                    table_hbm.at[idx_v.at[j]],
                    rows_g.at[b, pl.ds(j * IDX_ROW, IDX_ROW)],
                    sems_g[b],
                ).wait()

        def out_slice(i):
            return out_hbm.at[pl.ds(base + i * CHUNK, CHUNK), pl.ds(0, D_MODEL)]

        def start_scatter(i, b):
            pltpu.async_copy(rows_o.at[b], out_slice(i), sems_o[b])

        def wait_scatter(b):
            pltpu.make_async_copy(rows_o.at[b], out_slice(0), sems_o[b]).wait()

        def scale(b):
            def body8(t, c):
                r0 = t * 8
                for u in range(8):
                    for j in range(D_MODEL // 16):
                        sl = pl.ds(j * 16, 16)
                        rows_o[b, r0 + u, sl] = rows_g[b, r0 + u, sl] * SCALE
                return c

            lax.fori_loop(0, CHUNK // 8, body8, 0)

        start_gather(0, 0)
        start_gather(1, 1)

        for i in (0, 1):
            b = i % 2
            wait_gather(b)
            scale(b)
            start_scatter(i, b)
            start_gather(i + 2, b)

        def steady(t, c):
            i0 = 2 + t * 2
            for b in (0, 1):
                i = i0 + b
                wait_gather(b)
                wait_scatter(b)
                scale(b)
                start_scatter(i, b)
                start_gather(i + 2, b)
            return c

        lax.fori_loop(0, (n_chunks - 4) // 2, steady, 0)

        for i in (n_chunks - 2, n_chunks - 1):
            b = i % 2
            wait_gather(b)
            wait_scatter(b)
            scale(b)
            start_scatter(i, b)

        wait_scatter(0)
        wait_scatter(1)

    return k(table2, idx2d)


def kernel(x, table):
    b, t = x.shape
    B = b * t
    idx2d = (x.astype(jnp.int32) * 2).reshape(B // IDX_ROW, IDX_ROW)
    table2 = jnp.pad(table, ((0, 0), (0, D_MODEL))).reshape(-1, D_MODEL)
    out = _gather_scale(table2, idx2d, B)
    return out.reshape(b, t, OUT_W)[:, :, :D_MODEL]
